# Initial kernel scaffold; baseline (speedup 1.0000x reference)
import jax
import jax.numpy as jnp
from jax.experimental import pallas as pl


def kernel(x, emb_weight, ln_weight, ln_bias):
    raise NotImplementedError("write your pallas kernel here")


# speedup vs baseline: 2.7433x; 2.7433x over previous
import functools

import jax
import jax.numpy as jnp
from jax import lax
from jax.experimental import pallas as pl
from jax.experimental.pallas import tpu as pltpu
from jax.experimental.pallas import tpu_sc as plsc

NUM_EMB = 42
EMB_DIM = 16
B = 16384
L = 200
TOKENS = B * L

NUM_CORES = 2
NUM_SUBCORES = 16
NW = NUM_CORES * NUM_SUBCORES
TOK_PER_W = TOKENS // NW
CHUNK = 6400
NUM_CHUNKS = TOK_PER_W // CHUNK


def _table_body(emb_ref, lnw_ref, lnb_ref, out_ref):
    col = lax.broadcasted_iota(jnp.int32, (NUM_EMB, EMB_DIM), 1).astype(jnp.float32)
    e = emb_ref[:, :] + col
    mean = jnp.mean(e, axis=1, keepdims=True)
    d = e - mean
    var = jnp.mean(d * d, axis=1, keepdims=True)
    y = d * lax.rsqrt(var + 1e-5) * lnw_ref[:, :] + lnb_ref[:, :]
    out_ref[:, :] = jax.nn.sigmoid(y)


def _transform_table(emb_weight, ln_weight, ln_bias):
    return pl.pallas_call(
        _table_body,
        out_shape=jax.ShapeDtypeStruct((NUM_EMB, EMB_DIM), jnp.float32),
    )(emb_weight, ln_weight.reshape(1, EMB_DIM), ln_bias.reshape(1, EMB_DIM))


_mesh = plsc.VectorSubcoreMesh(
    core_axis_name="c", subcore_axis_name="s",
    num_cores=NUM_CORES, num_subcores=NUM_SUBCORES,
)


@functools.partial(
    pl.kernel,
    out_type=jax.ShapeDtypeStruct((TOKENS, EMB_DIM), jnp.float32),
    mesh=_mesh,
    scratch_types=[
        pltpu.VMEM((CHUNK,), jnp.int32),
        pltpu.VMEM((CHUNK, EMB_DIM), jnp.float32),
        pltpu.SemaphoreType.DMA,
    ],
    compiler_params=pltpu.CompilerParams(use_tc_tiling_on_sc=False),
)
def _gather_kernel(table_hbm, idx_hbm, out_hbm, idx_v, rows_v, sem):
    wid = lax.axis_index("s") * NUM_CORES + lax.axis_index("c")
    base = wid * TOK_PER_W

    def chunk_body(k):
        off = base + k * CHUNK
        pltpu.sync_copy(idx_hbm.at[pl.ds(off, CHUNK)], idx_v)
        pltpu.async_copy(table_hbm.at[idx_v], rows_v, sem).wait()
        pltpu.sync_copy(rows_v, out_hbm.at[pl.ds(off, CHUNK)])

    pl.loop(0, NUM_CHUNKS)(chunk_body)


@jax.jit
def kernel(x, emb_weight, ln_weight, ln_bias):
    table = _transform_table(emb_weight, ln_weight, ln_bias)
    flat_idx = x.reshape(TOKENS).astype(jnp.int32)
    out = _gather_kernel(table, flat_idx)
    return out.reshape(B, L, EMB_DIM)


# speedup vs baseline: 4.9318x; 1.7978x over previous
import functools

import jax
import jax.numpy as jnp
from jax import lax
from jax.experimental import pallas as pl
from jax.experimental.pallas import tpu as pltpu
from jax.experimental.pallas import tpu_sc as plsc

NUM_EMB = 42
EMB_DIM = 16
B = 16384
L = 200
TOKENS = B * L

NUM_CORES = 2
NUM_SUBCORES = 16
NW = NUM_CORES * NUM_SUBCORES
TOK_PER_W = TOKENS // NW
CHUNK = 3200
NUM_CHUNKS = TOK_PER_W // CHUNK


def _table_body(emb_ref, lnw_ref, lnb_ref, out_ref):
    col = lax.broadcasted_iota(jnp.int32, (NUM_EMB, EMB_DIM), 1).astype(jnp.float32)
    e = emb_ref[:, :] + col
    mean = jnp.mean(e, axis=1, keepdims=True)
    d = e - mean
    var = jnp.mean(d * d, axis=1, keepdims=True)
    y = d * lax.rsqrt(var + 1e-5) * lnw_ref[:, :] + lnb_ref[:, :]
    out_ref[:, :] = jax.nn.sigmoid(y)


def _transform_table(emb_weight, ln_weight, ln_bias):
    return pl.pallas_call(
        _table_body,
        out_shape=jax.ShapeDtypeStruct((NUM_EMB, EMB_DIM), jnp.float32),
    )(emb_weight, ln_weight.reshape(1, EMB_DIM), ln_bias.reshape(1, EMB_DIM))


_mesh = plsc.VectorSubcoreMesh(
    core_axis_name="c", subcore_axis_name="s",
    num_cores=NUM_CORES, num_subcores=NUM_SUBCORES,
)


@functools.partial(
    pl.kernel,
    out_type=jax.ShapeDtypeStruct((TOKENS, EMB_DIM), jnp.float32),
    mesh=_mesh,
    scratch_types=[
        pltpu.VMEM((NUM_EMB, EMB_DIM), jnp.float32),
        pltpu.VMEM((2, CHUNK), jnp.int32),
        pltpu.VMEM((2, CHUNK, EMB_DIM), jnp.float32),
        pltpu.SemaphoreType.DMA,
        pltpu.SemaphoreType.DMA,
    ],
    compiler_params=pltpu.CompilerParams(
        use_tc_tiling_on_sc=False, needs_layout_passes=False
    ),
)
def _gather_kernel(table_hbm, idx_hbm, out_hbm, table_v, idx2, rows2, sem_a, sem_c):
    wid = lax.axis_index("s") * NUM_CORES + lax.axis_index("c")
    base = wid * TOK_PER_W

    pltpu.sync_copy(table_hbm, table_v)

    def idx_src(k):
        return idx_hbm.at[pl.ds(base + k * CHUNK, CHUNK)]

    def out_dst(k):
        return out_hbm.at[pl.ds(base + k * CHUNK, CHUNK)]

    pltpu.async_copy(idx_src(0), idx2.at[0], sem_a)

    def outer(g):
        for b in (0, 1):
            k = g + b
            pltpu.make_async_copy(idx_src(k), idx2.at[b], sem_a).wait()

            @pl.when(k + 1 < NUM_CHUNKS)
            def _():
                pltpu.async_copy(idx_src(k + 1), idx2.at[1 - b], sem_a)

            @pl.when(k >= 2)
            def _():
                pltpu.make_async_copy(rows2.at[b], out_dst(k - 2), sem_c).wait()

            def compute(i):
                v = idx2[b, pl.ds(i * 16, 16)]
                tok = lax.iota(jnp.int32, 16) + i * 16
                for j in range(EMB_DIM):
                    colj = jnp.full((16,), j, jnp.int32)
                    got = plsc.load_gather(table_v, [v, colj])
                    plsc.store_scatter(rows2.at[b], [tok, colj], got)

            pl.loop(0, CHUNK // 16)(compute)
            pltpu.async_copy(rows2.at[b], out_dst(k), sem_c)

    pl.loop(0, NUM_CHUNKS, step=2)(outer)
    pltpu.make_async_copy(rows2.at[0], out_dst(NUM_CHUNKS - 2), sem_c).wait()
    pltpu.make_async_copy(rows2.at[1], out_dst(NUM_CHUNKS - 1), sem_c).wait()


@jax.jit
def kernel(x, emb_weight, ln_weight, ln_bias):
    table = _transform_table(emb_weight, ln_weight, ln_bias)
    flat_idx = x.reshape(TOKENS).astype(jnp.int32)
    out = _gather_kernel(table, flat_idx)
    return out.reshape(B, L, EMB_DIM)


# speedup vs baseline: 4.9378x; 1.0012x over previous
import functools

import jax
import jax.numpy as jnp
from jax import lax
from jax.experimental import pallas as pl
from jax.experimental.pallas import tpu as pltpu
from jax.experimental.pallas import tpu_sc as plsc

NUM_EMB = 42
EMB_DIM = 16
B = 16384
L = 200
TOKENS = B * L

NUM_CORES = 2
NUM_SUBCORES = 16
NW = NUM_CORES * NUM_SUBCORES
TOK_PER_W = TOKENS // NW
CHUNK = 3200
NUM_CHUNKS = TOK_PER_W // CHUNK


def _table_body(emb_ref, lnw_ref, lnb_ref, out_ref):
    col = lax.broadcasted_iota(jnp.int32, (NUM_EMB, EMB_DIM), 1).astype(jnp.float32)
    e = emb_ref[:, :] + col
    mean = jnp.mean(e, axis=1, keepdims=True)
    d = e - mean
    var = jnp.mean(d * d, axis=1, keepdims=True)
    y = d * lax.rsqrt(var + 1e-5) * lnw_ref[:, :] + lnb_ref[:, :]
    out_ref[:, :] = jax.nn.sigmoid(y)


def _transform_table(emb_weight, ln_weight, ln_bias):
    return pl.pallas_call(
        _table_body,
        out_shape=jax.ShapeDtypeStruct((NUM_EMB, EMB_DIM), jnp.float32),
    )(emb_weight, ln_weight.reshape(1, EMB_DIM), ln_bias.reshape(1, EMB_DIM))


_mesh = plsc.VectorSubcoreMesh(
    core_axis_name="c", subcore_axis_name="s",
    num_cores=NUM_CORES, num_subcores=NUM_SUBCORES,
)


CB = CHUNK // L
B_PER_W = B // NW


@functools.partial(
    pl.kernel,
    out_type=jax.ShapeDtypeStruct((B, L, EMB_DIM), jnp.float32),
    mesh=_mesh,
    scratch_types=[
        pltpu.VMEM((NUM_EMB, EMB_DIM), jnp.float32),
        pltpu.VMEM((2, CHUNK), jnp.int32),
        pltpu.VMEM((2, CB, L, EMB_DIM), jnp.float32),
        pltpu.SemaphoreType.DMA,
        pltpu.SemaphoreType.DMA,
    ],
    compiler_params=pltpu.CompilerParams(
        use_tc_tiling_on_sc=False, needs_layout_passes=False
    ),
)
def _gather_kernel(table_hbm, idx_hbm, out_hbm, table_v, idx2, rows2, sem_a, sem_c):
    wid = lax.axis_index("s") * NUM_CORES + lax.axis_index("c")
    base = wid * TOK_PER_W

    pltpu.sync_copy(table_hbm, table_v)

    def idx_src(k):
        return idx_hbm.at[pl.ds(base + k * CHUNK, CHUNK)]

    def out_dst(k):
        return out_hbm.at[pl.ds(wid * B_PER_W + k * CB, CB)]

    pltpu.async_copy(idx_src(0), idx2.at[0], sem_a)

    def outer(g):
        for b in (0, 1):
            k = g + b
            pltpu.make_async_copy(idx_src(k), idx2.at[b], sem_a).wait()

            @pl.when(k + 1 < NUM_CHUNKS)
            def _():
                pltpu.async_copy(idx_src(k + 1), idx2.at[1 - b], sem_a)

            @pl.when(k >= 2)
            def _():
                pltpu.make_async_copy(rows2.at[b], out_dst(k - 2), sem_c).wait()

            def compute(i):
                v = idx2[b, pl.ds(i * 16, 16)]
                tok = lax.iota(jnp.int32, 16) + i * 16
                bat = tok // L
                pos = tok - bat * L
                for j in range(EMB_DIM):
                    colj = jnp.full((16,), j, jnp.int32)
                    got = plsc.load_gather(table_v, [v, colj])
                    plsc.store_scatter(rows2.at[b], [bat, pos, colj], got)

            pl.loop(0, CHUNK // 16)(compute)
            pltpu.async_copy(rows2.at[b], out_dst(k), sem_c)

    pl.loop(0, NUM_CHUNKS, step=2)(outer)
    pltpu.make_async_copy(rows2.at[0], out_dst(NUM_CHUNKS - 2), sem_c).wait()
    pltpu.make_async_copy(rows2.at[1], out_dst(NUM_CHUNKS - 1), sem_c).wait()


@jax.jit
def kernel(x, emb_weight, ln_weight, ln_bias):
    table = _transform_table(emb_weight, ln_weight, ln_bias)
    flat_idx = x.reshape(TOKENS).astype(jnp.int32)
    return _gather_kernel(table, flat_idx)


# speedup vs baseline: 4.9394x; 1.0003x over previous
import functools

import jax
import jax.numpy as jnp
from jax import lax
from jax.experimental import pallas as pl
from jax.experimental.layout import Format, Layout
from jax.experimental.pallas import tpu as pltpu
from jax.experimental.pallas import tpu_sc as plsc

NUM_EMB = 42
EMB_DIM = 16
B = 16384
L = 200
TOKENS = B * L

NUM_CORES = 2
NUM_SUBCORES = 16
NW = NUM_CORES * NUM_SUBCORES
TOK_PER_W = TOKENS // NW
CHUNK = 3200
NUM_CHUNKS = TOK_PER_W // CHUNK


def _table_body(emb_ref, lnw_ref, lnb_ref, out_ref):
    col = lax.broadcasted_iota(jnp.int32, (NUM_EMB, EMB_DIM), 1).astype(jnp.float32)
    e = emb_ref[:, :] + col
    mean = jnp.mean(e, axis=1, keepdims=True)
    d = e - mean
    var = jnp.mean(d * d, axis=1, keepdims=True)
    y = d * lax.rsqrt(var + 1e-5) * lnw_ref[:, :] + lnb_ref[:, :]
    out_ref[:, :] = jax.nn.sigmoid(y)


def _transform_table(emb_weight, ln_weight, ln_bias):
    return pl.pallas_call(
        _table_body,
        out_shape=jax.ShapeDtypeStruct((NUM_EMB, EMB_DIM), jnp.float32),
    )(emb_weight, ln_weight.reshape(1, EMB_DIM), ln_bias.reshape(1, EMB_DIM))


_mesh = plsc.VectorSubcoreMesh(
    core_axis_name="c", subcore_axis_name="s",
    num_cores=NUM_CORES, num_subcores=NUM_SUBCORES,
)


CB = CHUNK // L
B_PER_W = B // NW


@functools.partial(
    pl.kernel,
    out_type=jax.ShapeDtypeStruct((B, L, EMB_DIM), jnp.float32),
    mesh=_mesh,
    scratch_types=[
        pltpu.VMEM((NUM_EMB, EMB_DIM), jnp.float32),
        pltpu.VMEM((2, CHUNK), jnp.int32),
        pltpu.VMEM((2, CB, L, EMB_DIM), jnp.float32),
        pltpu.SemaphoreType.DMA,
        pltpu.SemaphoreType.DMA,
    ],
    compiler_params=pltpu.CompilerParams(
        use_tc_tiling_on_sc=False, needs_layout_passes=False
    ),
)
def _gather_kernel(table_hbm, idx_hbm, out_hbm, table_v, idx2, rows2, sem_a, sem_c):
    wid = lax.axis_index("s") * NUM_CORES + lax.axis_index("c")
    base = wid * TOK_PER_W

    pltpu.sync_copy(table_hbm, table_v)

    def idx_src(k):
        return idx_hbm.at[pl.ds(base + k * CHUNK, CHUNK)]

    def out_dst(k):
        return out_hbm.at[pl.ds(wid * B_PER_W + k * CB, CB)]

    pltpu.async_copy(idx_src(0), idx2.at[0], sem_a)

    def outer(g):
        for b in (0, 1):
            k = g + b
            pltpu.make_async_copy(idx_src(k), idx2.at[b], sem_a).wait()

            @pl.when(k + 1 < NUM_CHUNKS)
            def _():
                pltpu.async_copy(idx_src(k + 1), idx2.at[1 - b], sem_a)

            @pl.when(k >= 2)
            def _():
                pltpu.make_async_copy(rows2.at[b], out_dst(k - 2), sem_c).wait()

            def compute(i):
                v = idx2[b, pl.ds(i * 16, 16)]
                tok = lax.iota(jnp.int32, 16) + i * 16
                bat = tok // L
                pos = tok - bat * L
                for j in range(EMB_DIM):
                    colj = jnp.full((16,), j, jnp.int32)
                    got = plsc.load_gather(table_v, [v, colj])
                    plsc.store_scatter(rows2.at[b], [bat, pos, colj], got)

            pl.loop(0, CHUNK // 16)(compute)
            pltpu.async_copy(rows2.at[b], out_dst(k), sem_c)

    pl.loop(0, NUM_CHUNKS, step=2)(outer)
    pltpu.make_async_copy(rows2.at[0], out_dst(NUM_CHUNKS - 2), sem_c).wait()
    pltpu.make_async_copy(rows2.at[1], out_dst(NUM_CHUNKS - 1), sem_c).wait()


def _kernel_impl(x, emb_weight, ln_weight, ln_bias):
    table = _transform_table(emb_weight, ln_weight, ln_bias)
    flat_idx = x.reshape(TOKENS).astype(jnp.int32)
    return _gather_kernel(table, flat_idx)


_jitted = None


def kernel(x, emb_weight, ln_weight, ln_bias):
    global _jitted
    if _jitted is None:
        fmt = Format(
            Layout(major_to_minor=(0, 1, 2), tiling=((8, 16),)),
            jax.sharding.SingleDeviceSharding(jax.devices()[0]),
        )
        _jitted = jax.jit(_kernel_impl, out_shardings=fmt)
    return _jitted(x, emb_weight, ln_weight, ln_bias)
